# baseline (speedup 1.0000x reference)
import functools

import jax
import jax.numpy as jnp
from jax import lax
from jax.experimental import pallas as pl
from jax.experimental.pallas import tpu as pltpu
from jax.experimental.pallas import tpu_sc as plsc


def kernel(indices, logits, feats):
    B = indices.shape[0]
    DL = logits.shape[1]
    DF = feats.shape[1]

    info = plsc.get_sparse_core_info()
    NC, NS = info.num_cores, info.num_subcores
    NW = NC * NS
    b_per_w = B // NW
    CH = 64
    n_ch = b_per_w // CH

    DM = (DL // 128) * 128
    DT = DL - DM

    idx32 = indices.astype(jnp.int32)
    idx3 = idx32.reshape(NW, n_ch, CH)
    mesh = plsc.VectorSubcoreMesh(core_axis_name="c", subcore_axis_name="s")

    @functools.partial(
        pl.kernel,
        mesh=mesh,
        out_type=(
            jax.ShapeDtypeStruct((B, DL), jnp.float32),
            jax.ShapeDtypeStruct((B, DF), jnp.float32),
        ),
        scratch_types=[
            pltpu.VMEM((n_ch, CH), jnp.int32),
            pltpu.VMEM((CH, DM), jnp.float32),
            pltpu.VMEM((CH, DF), jnp.float32),
            pltpu.SemaphoreType.DMA,
        ],
    )
    def gather_rows(idx_hbm, logits_hbm, feats_hbm, out_l_hbm, out_f_hbm,
                    idx_v, lmain_v, frows_v, sem):
        wid = lax.axis_index("s") * NC + lax.axis_index("c")
        base = wid * b_per_w
        pltpu.sync_copy(idx_hbm.at[wid], idx_v)

        def body(j, carry):
            row0 = base + j * CH
            pltpu.async_copy(
                logits_hbm.at[idx_v.at[j], pl.ds(0, DM)], lmain_v, sem).wait()
            pltpu.sync_copy(lmain_v, out_l_hbm.at[pl.ds(row0, CH), pl.ds(0, DM)])
            pltpu.async_copy(feats_hbm.at[idx_v.at[j]], frows_v, sem).wait()
            pltpu.sync_copy(frows_v, out_f_hbm.at[pl.ds(row0, CH)])
            return carry

        lax.fori_loop(0, n_ch, body, 0)

    out_l, out_f = gather_rows(idx3, logits, feats)

    starts = jnp.stack(
        [idx32, jnp.full_like(idx32, DM)], axis=1)
    dnums = lax.GatherDimensionNumbers(
        offset_dims=(1,), collapsed_slice_dims=(0,), start_index_map=(0, 1))
    tail = lax.gather(
        logits, starts, dnums, slice_sizes=(1, DT),
        mode=lax.GatherScatterMode.PROMISE_IN_BOUNDS)
    out_l = lax.dynamic_update_slice(out_l, tail, (0, DM))
    return (out_l, out_f)


# speedup vs baseline: 144.3715x; 144.3715x over previous
import functools

import jax
import jax.numpy as jnp
from jax import lax
from jax.experimental import pallas as pl
from jax.experimental.pallas import tpu as pltpu
from jax.experimental.pallas import tpu_sc as plsc


def kernel(indices, logits, feats):
    B = indices.shape[0]
    DL = logits.shape[1]
    DF = feats.shape[1]

    info = plsc.get_sparse_core_info()
    NC, NS = info.num_cores, info.num_subcores
    NW = NC * NS
    b_per_w = B // NW
    CH = 64
    n_ch = b_per_w // CH

    DM = (DL // 128) * 128
    DT = DL - DM

    idx32 = indices.astype(jnp.int32)
    idx3 = idx32.reshape(NW, n_ch, CH)
    tail_pad = jnp.pad(logits[:, DM:], ((0, 0), (0, 128 - DT)))

    mesh = plsc.VectorSubcoreMesh(core_axis_name="c", subcore_axis_name="s")

    @functools.partial(
        pl.kernel,
        mesh=mesh,
        out_type=(
            jax.ShapeDtypeStruct((B, DL), jnp.float32),
            jax.ShapeDtypeStruct((B, 128), jnp.float32),
            jax.ShapeDtypeStruct((B, DF), jnp.float32),
        ),
        scratch_types=[
            pltpu.VMEM((n_ch, CH), jnp.int32),
            pltpu.VMEM((CH, DM), jnp.float32),
            pltpu.VMEM((CH, 128), jnp.float32),
            pltpu.VMEM((CH, DF), jnp.float32),
            pltpu.SemaphoreType.DMA,
        ],
    )
    def gather_rows(idx_hbm, logits_hbm, tail_hbm, feats_hbm,
                    out_l_hbm, out_t_hbm, out_f_hbm,
                    idx_v, lmain_v, ltail_v, frows_v, sem):
        wid = lax.axis_index("s") * NC + lax.axis_index("c")
        base = wid * b_per_w
        pltpu.sync_copy(idx_hbm.at[wid], idx_v)

        def body(j, carry):
            row0 = base + j * CH
            pltpu.async_copy(
                logits_hbm.at[idx_v.at[j], pl.ds(0, DM)], lmain_v, sem).wait()
            pltpu.sync_copy(lmain_v, out_l_hbm.at[pl.ds(row0, CH), pl.ds(0, DM)])
            pltpu.async_copy(tail_hbm.at[idx_v.at[j]], ltail_v, sem).wait()
            pltpu.sync_copy(ltail_v, out_t_hbm.at[pl.ds(row0, CH)])
            pltpu.async_copy(feats_hbm.at[idx_v.at[j]], frows_v, sem).wait()
            pltpu.sync_copy(frows_v, out_f_hbm.at[pl.ds(row0, CH)])
            return carry

        lax.fori_loop(0, n_ch, body, 0)

    out_l, out_t, out_f = gather_rows(idx3, logits, tail_pad, feats)
    out_l = lax.dynamic_update_slice(out_l, out_t[:, :DT], (0, DM))
    return (out_l, out_f)


# speedup vs baseline: 271.8492x; 1.8830x over previous
import functools

import jax
import jax.numpy as jnp
from jax import lax
from jax.experimental import pallas as pl
from jax.experimental.pallas import tpu as pltpu
from jax.experimental.pallas import tpu_sc as plsc


def kernel(indices, logits, feats):
    B = indices.shape[0]
    NR, DL = logits.shape
    DF = feats.shape[1]

    info = plsc.get_sparse_core_info()
    NC, NS = info.num_cores, info.num_subcores
    NW = NC * NS
    b_per_w = B // NW
    FCH = 16
    n_fch = b_per_w // FCH
    OC = 4096
    n_oc = B // OC

    idx32 = indices.astype(jnp.int32)
    lT = logits.T

    mesh = plsc.VectorSubcoreMesh(core_axis_name="c", subcore_axis_name="s")

    @functools.partial(
        pl.kernel,
        mesh=mesh,
        compiler_params=pltpu.CompilerParams(needs_layout_passes=False),
        out_type=(
            jax.ShapeDtypeStruct((DL, B), jnp.float32),
            jax.ShapeDtypeStruct((B, DF), jnp.float32),
        ),
        scratch_types=[
            pltpu.VMEM((B,), jnp.int32),
            pltpu.VMEM((NR,), jnp.float32),
            pltpu.VMEM((OC,), jnp.float32),
            pltpu.VMEM((FCH, DF), jnp.float32),
            pltpu.SemaphoreType.DMA,
            pltpu.SemaphoreType.DMA,
        ],
    )
    def gather_all(idx_hbm, lT_hbm, feats_hbm, oT_hbm, of_hbm,
                   idx_v, row_v, outc_v, fv, sem, fsem):
        wid = lax.axis_index("s") * NC + lax.axis_index("c")
        pltpu.sync_copy(idx_hbm, idx_v)

        fbase = wid * b_per_w

        def fbody(j, c):
            r0 = fbase + j * FCH
            pltpu.async_copy(
                feats_hbm.at[idx_v.at[pl.ds(r0, FCH)]], fv, fsem).wait()
            pltpu.sync_copy(fv, of_hbm.at[pl.ds(r0, FCH)])
            return c

        lax.fori_loop(0, n_fch, fbody, 0)

        n_rows = (DL - 1 - wid) // NW + 1

        def rbody(t, c):
            r = wid + NW * t
            pltpu.async_copy(lT_hbm.at[r], row_v, sem).wait()

            def cbody(k, c2):
                def vbody(v, c3):
                    iis = []
                    for u in range(8):
                        o = pl.multiple_of(k * OC + v * 128 + u * 16, 16)
                        iis.append(idx_v[pl.ds(o, 16)])
                    gs = [plsc.load_gather(row_v, [ii]) for ii in iis]
                    for u in range(8):
                        o = pl.multiple_of(v * 128 + u * 16, 16)
                        outc_v[pl.ds(o, 16)] = gs[u]
                    return c3

                lax.fori_loop(0, OC // 128, vbody, 0)
                pltpu.sync_copy(outc_v, oT_hbm.at[r, pl.ds(k * OC, OC)])
                return c2

            lax.fori_loop(0, n_oc, cbody, 0)
            return c

        lax.fori_loop(0, n_rows, rbody, 0)

    oT, out_f = gather_all(idx32, lT, feats)
    return (oT.T, out_f)


# speedup vs baseline: 295.8759x; 1.0884x over previous
import functools

import jax
import jax.numpy as jnp
from jax import lax
from jax.experimental import pallas as pl
from jax.experimental.pallas import tpu as pltpu
from jax.experimental.pallas import tpu_sc as plsc


def kernel(indices, logits, feats):
    B = indices.shape[0]
    NR, DL = logits.shape
    DF = feats.shape[1]

    info = plsc.get_sparse_core_info()
    NC, NS = info.num_cores, info.num_subcores
    NW = NC * NS
    b_per_w = B // NW
    FCH = 16
    n_fch = b_per_w // FCH
    OC = 2048
    n_oc = B // OC
    NH = NR // 2

    idx32 = indices.astype(jnp.int32)
    lT = logits.T

    mesh = plsc.VectorSubcoreMesh(core_axis_name="c", subcore_axis_name="s")

    @functools.partial(
        pl.kernel,
        mesh=mesh,
        compiler_params=pltpu.CompilerParams(needs_layout_passes=False),
        out_type=(
            jax.ShapeDtypeStruct((DL, B), jnp.float32),
            jax.ShapeDtypeStruct((B, DF), jnp.float32),
        ),
        scratch_types=[
            pltpu.VMEM((B,), jnp.int32),
            pltpu.VMEM((NR,), jnp.float32),
            pltpu.VMEM((OC,), jnp.float32),
            pltpu.VMEM((OC,), jnp.float32),
            pltpu.VMEM((FCH, DF), jnp.float32),
            pltpu.SemaphoreType.DMA,
            pltpu.SemaphoreType.DMA,
            pltpu.SemaphoreType.DMA,
            pltpu.SemaphoreType.DMA,
        ],
    )
    def gather_all(idx_hbm, lT_hbm, feats_hbm, oT_hbm, of_hbm,
                   idx_v, row_v, outc0_v, outc1_v, fv,
                   sem, fsem, osem0, osem1):
        wid = lax.axis_index("s") * NC + lax.axis_index("c")
        pltpu.sync_copy(idx_hbm, idx_v)

        fbase = wid * b_per_w

        def fbody(j, c):
            r0 = fbase + j * FCH
            pltpu.async_copy(
                feats_hbm.at[idx_v.at[pl.ds(r0, FCH)]], fv, fsem).wait()
            pltpu.sync_copy(fv, of_hbm.at[pl.ds(r0, FCH)])
            return c

        lax.fori_loop(0, n_fch, fbody, 0)

        n_rows = (DL - 1 - wid) // NW + 1
        obufs = (outc0_v, outc1_v)
        osems = (osem0, osem1)

        def rbody(t, c):
            r = wid + NW * t
            pltpu.async_copy(lT_hbm.at[r], row_v, sem).wait()

            def cbody(k2, c2):
                for h in range(2):
                    k = k2 * 2 + h
                    ob, osem = obufs[h], osems[h]

                    @pl.when((t > 0) | (k2 > 0))
                    def _():
                        pltpu.make_async_copy(
                            ob, oT_hbm.at[r, pl.ds(0, OC)], osem).wait()

                    def vbody(v, c3):
                        iis = []
                        for u in range(8):
                            o = pl.multiple_of(k * OC + v * 128 + u * 16, 16)
                            iis.append(idx_v[pl.ds(o, 16)])
                        gs = [plsc.load_gather(row_v, [ii]) for ii in iis]
                        for u in range(8):
                            o = pl.multiple_of(v * 128 + u * 16, 16)
                            ob[pl.ds(o, 16)] = gs[u]
                        return c3

                    lax.fori_loop(0, OC // 128, vbody, 0)
                    pltpu.async_copy(ob, oT_hbm.at[r, pl.ds(k * OC, OC)], osem)
                return c2

            lax.fori_loop(0, n_oc // 2, cbody, 0)
            return c

        lax.fori_loop(0, n_rows, rbody, 0)
        pltpu.make_async_copy(
            outc0_v, oT_hbm.at[wid, pl.ds(0, OC)], osem0).wait()
        pltpu.make_async_copy(
            outc1_v, oT_hbm.at[wid, pl.ds(0, OC)], osem1).wait()

    oT, out_f = gather_all(idx32, lT, feats)
    return (oT.T, out_f)
